# baseline (speedup 1.0000x reference)
import jax
import jax.numpy as jnp
from jax.experimental import pallas as pl
from jax.experimental.pallas import tpu as pltpu

_NSUB = 2


def _project_chunk(y_ref, wkt_ref, bk_ref, wvt_ref, bv_ref, k_scr, v_scr, i, ch):
    yb = y_ref[0]
    k_scr[pl.ds(i * ch, ch), :] = jnp.dot(
        yb, wkt_ref[...], preferred_element_type=jnp.float32) + bk_ref[...]
    v_scr[pl.ds(i * ch, ch), :] = jnp.dot(
        yb, wvt_ref[...], preferred_element_type=jnp.float32) + bv_ref[...]


def _attend(x_ref, wqt_ref, bq_ref, k_scr, v_scr, o_ref):
    xb = x_ref[0]
    sub = xb.shape[0] // _NSUB
    for h in range(_NSUB):
        xh = xb[h * sub:(h + 1) * sub]
        q = jnp.dot(xh, wqt_ref[...],
                    preferred_element_type=jnp.float32) + bq_ref[...]
        s = jax.lax.dot_general(q, k_scr[...], (((1,), (1,)), ((), ())),
                                preferred_element_type=jnp.float32)
        m = jnp.max(s, axis=-1, keepdims=True)
        p = jnp.exp(s - m)
        l = jnp.sum(p, axis=-1, keepdims=True)
        o = jnp.dot(p, v_scr[...], preferred_element_type=jnp.float32)
        o_ref[0, h * sub:(h + 1) * sub] = o / l + xh


def _make_kernel(nb, ch):
    def _fused_kernel(x_ref, y_ref, wqt_ref, bq_ref, wkt_ref, bk_ref,
                      wvt_ref, bv_ref, o_ref, k0, v0, k1, v1):
        bb = pl.program_id(0)
        i = pl.program_id(1)
        wpar = jax.lax.rem(bb, 2)
        rpar = jax.lax.rem(bb + 1, 2)

        @pl.when((bb < nb) & (wpar == 0))
        def _():
            _project_chunk(y_ref, wkt_ref, bk_ref, wvt_ref, bv_ref,
                           k0, v0, i, ch)

        @pl.when((bb < nb) & (wpar == 1))
        def _():
            _project_chunk(y_ref, wkt_ref, bk_ref, wvt_ref, bv_ref,
                           k1, v1, i, ch)

        @pl.when((bb > 0) & (rpar == 0))
        def _():
            _attend(x_ref, wqt_ref, bq_ref, k0, v0, o_ref)

        @pl.when((bb > 0) & (rpar == 1))
        def _():
            _attend(x_ref, wqt_ref, bq_ref, k1, v1, o_ref)

    return _fused_kernel


def kernel(x, y, Wq, bq, Wk, bk, Wv, bv):
    B, SX, D = x.shape
    SY = y.shape[1]
    ch = min(512, SX)
    nq = SX // ch

    wqt = Wq.T
    wkt = Wk.T
    wvt = Wv.T
    bq2 = bq.reshape(1, D)
    bk2 = bk.reshape(1, D)
    bv2 = bv.reshape(1, D)

    def xi(bb, i):
        return (jnp.maximum(bb - 1, 0), jnp.where(bb == 0, 0, i), 0)

    def yi(bb, i):
        return (jnp.minimum(bb, B - 1), jnp.where(bb == B, 0, i), 0)

    out = pl.pallas_call(
        _make_kernel(B, ch),
        grid=(B + 1, nq),
        in_specs=[
            pl.BlockSpec((1, ch, D), xi),
            pl.BlockSpec((1, ch, D), yi),
            pl.BlockSpec((D, D), lambda bb, i: (0, 0)),
            pl.BlockSpec((1, D), lambda bb, i: (0, 0)),
            pl.BlockSpec((D, D), lambda bb, i: (0, 0)),
            pl.BlockSpec((1, D), lambda bb, i: (0, 0)),
            pl.BlockSpec((D, D), lambda bb, i: (0, 0)),
            pl.BlockSpec((1, D), lambda bb, i: (0, 0)),
        ],
        out_specs=pl.BlockSpec((1, ch, D), xi),
        out_shape=jax.ShapeDtypeStruct((B, SX, D), jnp.float32),
        scratch_shapes=[
            pltpu.VMEM((SY, D), jnp.float32),
            pltpu.VMEM((SY, D), jnp.float32),
            pltpu.VMEM((SY, D), jnp.float32),
            pltpu.VMEM((SY, D), jnp.float32),
        ],
    )(x, y, wqt, bq2, wkt, bk2, wvt, bv2)
    return out


# speedup vs baseline: 1.0253x; 1.0253x over previous
import jax
import jax.numpy as jnp
from jax.experimental import pallas as pl
from jax.experimental.pallas import tpu as pltpu

_CH = 1024
_NSUB = 4


def _fused_kernel(x_ref, y_ref, wqt_ref, bq_ref, wkt_ref, bk_ref,
                  wvt_ref, bv_ref, o_ref, k_scr, v_scr):
    i = pl.program_id(1)

    @pl.when(i == 0)
    def _project_kv():
        yb = y_ref[0]
        k_scr[...] = jnp.dot(yb, wkt_ref[...],
                             preferred_element_type=jnp.float32) + bk_ref[...]
        v_scr[...] = jnp.dot(yb, wvt_ref[...],
                             preferred_element_type=jnp.float32) + bv_ref[...]

    @pl.when(i > 0)
    def _attend():
        xb = x_ref[0]
        rows = xb.shape[0]
        sub = rows // _NSUB
        for h in range(_NSUB):
            xh = xb[h * sub:(h + 1) * sub]
            q = jnp.dot(xh, wqt_ref[...],
                        preferred_element_type=jnp.float32) + bq_ref[...]
            s = jax.lax.dot_general(q, k_scr[...], (((1,), (1,)), ((), ())),
                                    preferred_element_type=jnp.float32)
            m = jnp.max(s, axis=-1, keepdims=True)
            p = jnp.exp(s - m)
            l = jnp.sum(p, axis=-1, keepdims=True)
            o = jnp.dot(p, v_scr[...],
                        preferred_element_type=jnp.float32)
            o_ref[0, h * sub:(h + 1) * sub] = o / l + xh


def kernel(x, y, Wq, bq, Wk, bk, Wv, bv):
    B, SX, D = x.shape
    SY = y.shape[1]
    ch = min(_CH, SX)
    nq = SX // ch

    wqt = Wq.T
    wkt = Wk.T
    wvt = Wv.T
    bq2 = bq.reshape(1, D)
    bk2 = bk.reshape(1, D)
    bv2 = bv.reshape(1, D)

    def qi(b, i):
        return (b, jnp.maximum(i - 1, 0), 0)

    out = pl.pallas_call(
        _fused_kernel,
        grid=(B, nq + 1),
        in_specs=[
            pl.BlockSpec((1, ch, D), qi),
            pl.BlockSpec((1, SY, D), lambda b, i: (b, 0, 0)),
            pl.BlockSpec((D, D), lambda b, i: (0, 0)),
            pl.BlockSpec((1, D), lambda b, i: (0, 0)),
            pl.BlockSpec((D, D), lambda b, i: (0, 0)),
            pl.BlockSpec((1, D), lambda b, i: (0, 0)),
            pl.BlockSpec((D, D), lambda b, i: (0, 0)),
            pl.BlockSpec((1, D), lambda b, i: (0, 0)),
        ],
        out_specs=pl.BlockSpec((1, ch, D), qi),
        out_shape=jax.ShapeDtypeStruct((B, SX, D), jnp.float32),
        scratch_shapes=[
            pltpu.VMEM((SY, D), jnp.float32),
            pltpu.VMEM((SY, D), jnp.float32),
        ],
    )(x, y, wqt, bq2, wkt, bk2, wvt, bv2)
    return out


# speedup vs baseline: 1.0573x; 1.0312x over previous
import jax
import jax.numpy as jnp
from jax.experimental import pallas as pl
from jax.experimental.pallas import tpu as pltpu

_CH = 1024
_NSUB = 2


def _fused_kernel(x_ref, y_ref, wqt_ref, bq_ref, wkt_ref, bk_ref,
                  wvt_ref, bv_ref, o_ref, k_scr, v_scr):
    i = pl.program_id(1)

    @pl.when(i == 0)
    def _project_kv():
        yb = y_ref[0]
        k_scr[...] = jnp.dot(yb, wkt_ref[...],
                             preferred_element_type=jnp.float32) + bk_ref[...]
        v_scr[...] = jnp.dot(yb, wvt_ref[...],
                             preferred_element_type=jnp.float32) + bv_ref[...]

    @pl.when(i > 0)
    def _attend():
        xb = x_ref[0]
        rows = xb.shape[0]
        sub = rows // _NSUB
        for h in range(_NSUB):
            xh = xb[h * sub:(h + 1) * sub]
            q = jnp.dot(xh, wqt_ref[...],
                        preferred_element_type=jnp.float32) + bq_ref[...]
            s = jax.lax.dot_general(q, k_scr[...], (((1,), (1,)), ((), ())),
                                    preferred_element_type=jnp.float32)
            m = jnp.max(s, axis=-1, keepdims=True)
            p = jnp.exp(s - m)
            l = jnp.sum(p, axis=-1, keepdims=True)
            o = jnp.dot(p, v_scr[...],
                        preferred_element_type=jnp.float32)
            o_ref[0, h * sub:(h + 1) * sub] = o / l + xh


def kernel(x, y, Wq, bq, Wk, bk, Wv, bv):
    B, SX, D = x.shape
    SY = y.shape[1]
    ch = min(_CH, SX)
    nq = SX // ch

    wqt = Wq.T
    wkt = Wk.T
    wvt = Wv.T
    bq2 = bq.reshape(1, D)
    bk2 = bk.reshape(1, D)
    bv2 = bv.reshape(1, D)

    def qi(b, i):
        return (b, jnp.maximum(i - 1, 0), 0)

    out = pl.pallas_call(
        _fused_kernel,
        grid=(B, nq + 1),
        in_specs=[
            pl.BlockSpec((1, ch, D), qi),
            pl.BlockSpec((1, SY, D), lambda b, i: (b, 0, 0)),
            pl.BlockSpec((D, D), lambda b, i: (0, 0)),
            pl.BlockSpec((1, D), lambda b, i: (0, 0)),
            pl.BlockSpec((D, D), lambda b, i: (0, 0)),
            pl.BlockSpec((1, D), lambda b, i: (0, 0)),
            pl.BlockSpec((D, D), lambda b, i: (0, 0)),
            pl.BlockSpec((1, D), lambda b, i: (0, 0)),
        ],
        out_specs=pl.BlockSpec((1, ch, D), qi),
        out_shape=jax.ShapeDtypeStruct((B, SX, D), jnp.float32),
        scratch_shapes=[
            pltpu.VMEM((SY, D), jnp.float32),
            pltpu.VMEM((SY, D), jnp.float32),
        ],
    )(x, y, wqt, bq2, wkt, bk2, wvt, bv2)
    return out
